# baseline (speedup 1.0000x reference)
import jax
import jax.numpy as jnp
from jax import lax
from jax.experimental import pallas as pl
from jax.experimental.pallas import tpu as pltpu, tpu_sc as plsc

NC = 2
NS = 16
L = 16
NW = NC * NS
GSUB = 80


def _mesh():
    return plsc.VectorSubcoreMesh(core_axis_name="c", subcore_axis_name="s",
                                  num_cores=NC, num_subcores=NS)


def _make_diff_kernel(n_links, n_nodes):
    OUT = n_links // NW
    B1 = 2000
    NB1 = OUT // B1
    assert OUT % B1 == 0 and B1 % L == 0

    def body(vel_hbm, head_hbm, tail_hbm, diff_out, vel_nodes, hblk, tblk, dblk, *_):
        c = lax.axis_index("c")
        s = lax.axis_index("s")
        wid = s * NC + c
        pltpu.sync_copy(vel_hbm.at[pl.ds(0, n_nodes)], vel_nodes)

        def p1_block(b, _):
            off = wid * OUT + b * B1
            pltpu.sync_copy(head_hbm.at[pl.ds(off, B1)], hblk)
            pltpu.sync_copy(tail_hbm.at[pl.ds(off, B1)], tblk)

            def p1_group(j, _):
                sl = pl.ds(j * L, L)
                hv = plsc.load_gather(vel_nodes, [hblk[sl]])
                tv = plsc.load_gather(vel_nodes, [tblk[sl]])
                dblk[sl] = hv - tv
                return 0

            lax.fori_loop(0, B1 // L, p1_group, 0)
            pltpu.sync_copy(dblk, diff_out.at[pl.ds(off, B1)])
            return 0

        lax.fori_loop(0, NB1, p1_block, 0)

    return pl.kernel(
        body,
        out_type=jax.ShapeDtypeStruct((n_links,), jnp.float32),
        mesh=_mesh(),
        scratch_types=(
            pltpu.VMEM((n_nodes,), jnp.float32),
            pltpu.VMEM((B1,), jnp.int32),
            pltpu.VMEM((B1,), jnp.int32),
            pltpu.VMEM((B1,), jnp.float32),
        ),
        compiler_params=pltpu.CompilerParams(needs_layout_passes=False),
        name="tvd_diff_sc",
    )


def _make_ratio_kernel(n_links):
    OUT = n_links // NW
    B2 = 2000
    NB2 = OUT // B2
    NR = B2 // GSUB
    LOADCH = n_links // NS
    assert OUT % B2 == 0 and B2 % GSUB == 0 and B2 % L == 0

    def body(vel_hbm, pflat_hbm, diff_hbm,
             upwind_out, ratio_out,
             velblk, pblk, ublk, u2d, g2d, dnblk, rblk, sem):
        c = lax.axis_index("c")
        s = lax.axis_index("s")
        wid = s * NC + c
        iota = lax.iota(jnp.int32, L)

        def p2_block(b, _):
            off = wid * OUT + b * B2
            pltpu.sync_copy(vel_hbm.at[pl.ds(off, B2)], velblk)
            pltpu.sync_copy(pflat_hbm.at[pl.ds(2 * off, 2 * B2)], pblk)

            def p2_upwind(j, _):
                v = velblk[pl.ds(j * L, L)]
                inv = (v <= 0.0).astype(jnp.int32)
                pidx = 2 * (j * L + iota) + inv
                u = plsc.load_gather(pblk, [pidx])
                ublk[pl.ds(j * L, L)] = u
                u2d[j // (GSUB // L), pl.ds((j % (GSUB // L)) * L, L)] = u
                return 0

            lax.fori_loop(0, B2 // L, p2_upwind, 0)
            pltpu.sync_copy(ublk, upwind_out.at[pl.ds(off, B2)])

            descs = [pltpu.async_copy(diff_hbm.at[u2d.at[r]], g2d.at[r], sem)
                     for r in range(NR)]
            pltpu.sync_copy(diff_hbm.at[pl.ds(off, B2)], dnblk)
            for d in descs:
                d.wait()

            def p2_ratio(j, _):
                sl = pl.ds(j * L, L)
                d = dnblk[sl]
                g = g2d[j // (GSUB // L), pl.ds((j % (GSUB // L)) * L, L)]
                u = ublk[sl]
                ok = (d != 0.0) & (u != -1)
                rblk[sl] = jnp.where(ok, g / d, 1.0)
                return 0

            lax.fori_loop(0, B2 // L, p2_ratio, 0)
            pltpu.sync_copy(rblk, ratio_out.at[pl.ds(off, B2)])
            return 0

        lax.fori_loop(0, NB2, p2_block, 0)

    return pl.kernel(
        body,
        out_type=(
            jax.ShapeDtypeStruct((n_links,), jnp.int32),
            jax.ShapeDtypeStruct((n_links,), jnp.float32),
        ),
        mesh=_mesh(),
        scratch_types=(
            pltpu.VMEM((B2,), jnp.float32),
            pltpu.VMEM((2 * B2,), jnp.int32),
            pltpu.VMEM((B2,), jnp.int32),
            pltpu.VMEM((NR, GSUB), jnp.int32),
            pltpu.VMEM((NR, GSUB), jnp.float32),
            pltpu.VMEM((B2,), jnp.float32),
            pltpu.VMEM((B2,), jnp.float32),
            pltpu.SemaphoreType.DMA,
        ),
        compiler_params=pltpu.CompilerParams(needs_layout_passes=False),
        name="tvd_ratio_sc",
    )


def kernel(velocity, field, parallel_links_at_link, node_at_link_head, node_at_link_tail):
    n_links = velocity.shape[0]
    n_nodes = field.shape[0]
    pflat = parallel_links_at_link.reshape(-1)
    diff = _make_diff_kernel(n_links, n_nodes)(velocity, node_at_link_head, node_at_link_tail)
    upwind, ratio = _make_ratio_kernel(n_links)(velocity, pflat, diff)
    return upwind, ratio


# speedup vs baseline: 8.2689x; 8.2689x over previous
import jax
import jax.numpy as jnp
from jax import lax
from jax.experimental import pallas as pl
from jax.experimental.pallas import tpu as pltpu, tpu_sc as plsc

NC = 2
NS = 16
L = 16
NW = NC * NS
GSUB = 80


def _mesh():
    return plsc.VectorSubcoreMesh(core_axis_name="c", subcore_axis_name="s",
                                  num_cores=NC, num_subcores=NS)


def _make_diff_kernel(n_links, n_nodes):
    OUT = n_links // NW
    B1 = 2000
    NB1 = OUT // B1
    assert OUT % B1 == 0 and B1 % L == 0

    def body(vel_hbm, head_hbm, tail_hbm, diff_out, vel_nodes, hblk, tblk, dblk, *_):
        c = lax.axis_index("c")
        s = lax.axis_index("s")
        wid = s * NC + c
        pltpu.sync_copy(vel_hbm.at[pl.ds(0, n_nodes)], vel_nodes)

        def p1_block(b, _):
            off = wid * OUT + b * B1
            pltpu.sync_copy(head_hbm.at[pl.ds(off, B1)], hblk)
            pltpu.sync_copy(tail_hbm.at[pl.ds(off, B1)], tblk)

            def p1_group(j, _):
                sl = pl.ds(j * L, L)
                hv = plsc.load_gather(vel_nodes, [hblk[sl]])
                tv = plsc.load_gather(vel_nodes, [tblk[sl]])
                dblk[sl] = hv - tv
                return 0

            lax.fori_loop(0, B1 // L, p1_group, 0)
            pltpu.sync_copy(dblk, diff_out.at[pl.ds(off, B1)])
            return 0

        lax.fori_loop(0, NB1, p1_block, 0)

    return pl.kernel(
        body,
        out_type=jax.ShapeDtypeStruct((n_links,), jnp.float32),
        mesh=_mesh(),
        scratch_types=(
            pltpu.VMEM((n_nodes,), jnp.float32),
            pltpu.VMEM((B1,), jnp.int32),
            pltpu.VMEM((B1,), jnp.int32),
            pltpu.VMEM((B1,), jnp.float32),
        ),
        compiler_params=pltpu.CompilerParams(needs_layout_passes=False),
        name="tvd_diff_sc",
    )


def _make_ratio_kernel(n_links):
    OUT = n_links // NW
    B2 = 2000
    NB2 = OUT // B2
    NR = B2 // GSUB
    LOADCH = n_links // NS
    assert OUT % B2 == 0 and B2 % GSUB == 0 and B2 % L == 0

    def body(vel_hbm, c0_hbm, c1_hbm, diff_hbm,
             upwind_out, ratio_out,
             velblk, c0blk, c1blk, ublk, u2d, g2d, dnblk, rblk, sem):
        c = lax.axis_index("c")
        s = lax.axis_index("s")
        wid = s * NC + c

        def p2_block(b, _):
            off = wid * OUT + b * B2
            pltpu.sync_copy(vel_hbm.at[pl.ds(off, B2)], velblk)
            pltpu.sync_copy(c0_hbm.at[pl.ds(off, B2)], c0blk)
            pltpu.sync_copy(c1_hbm.at[pl.ds(off, B2)], c1blk)

            def p2_upwind(j, _):
                sl = pl.ds(j * L, L)
                v = velblk[sl]
                u = jnp.where(v <= 0.0, c1blk[sl], c0blk[sl])
                ublk[sl] = u
                u2d[j // (GSUB // L), pl.ds((j % (GSUB // L)) * L, L)] = u
                return 0

            lax.fori_loop(0, B2 // L, p2_upwind, 0)
            pltpu.sync_copy(ublk, upwind_out.at[pl.ds(off, B2)])

            descs = [pltpu.async_copy(diff_hbm.at[u2d.at[r]], g2d.at[r], sem)
                     for r in range(NR)]
            pltpu.sync_copy(diff_hbm.at[pl.ds(off, B2)], dnblk)
            for d in descs:
                d.wait()

            def p2_ratio(j, _):
                sl = pl.ds(j * L, L)
                d = dnblk[sl]
                g = g2d[j // (GSUB // L), pl.ds((j % (GSUB // L)) * L, L)]
                u = ublk[sl]
                ok = (d != 0.0) & (u != -1)
                rblk[sl] = jnp.where(ok, g / d, 1.0)
                return 0

            lax.fori_loop(0, B2 // L, p2_ratio, 0)
            pltpu.sync_copy(rblk, ratio_out.at[pl.ds(off, B2)])
            return 0

        lax.fori_loop(0, NB2, p2_block, 0)

    return pl.kernel(
        body,
        out_type=(
            jax.ShapeDtypeStruct((n_links,), jnp.int32),
            jax.ShapeDtypeStruct((n_links,), jnp.float32),
        ),
        mesh=_mesh(),
        scratch_types=(
            pltpu.VMEM((B2,), jnp.float32),
            pltpu.VMEM((B2,), jnp.int32),
            pltpu.VMEM((B2,), jnp.int32),
            pltpu.VMEM((B2,), jnp.int32),
            pltpu.VMEM((NR, GSUB), jnp.int32),
            pltpu.VMEM((NR, GSUB), jnp.float32),
            pltpu.VMEM((B2,), jnp.float32),
            pltpu.VMEM((B2,), jnp.float32),
            pltpu.SemaphoreType.DMA,
        ),
        compiler_params=pltpu.CompilerParams(needs_layout_passes=False),
        name="tvd_ratio_sc",
    )


def kernel(velocity, field, parallel_links_at_link, node_at_link_head, node_at_link_tail):
    n_links = velocity.shape[0]
    n_nodes = field.shape[0]
    diff = _make_diff_kernel(n_links, n_nodes)(velocity, node_at_link_head, node_at_link_tail)
    c0 = parallel_links_at_link[:, 0]
    c1 = parallel_links_at_link[:, 1]
    upwind, ratio = _make_ratio_kernel(n_links)(velocity, c0, c1, diff)
    return upwind, ratio
